# baseline (speedup 1.0000x reference)
import jax
import jax.numpy as jnp
from jax.experimental import pallas as pl

KNN = 20
GRID = 32
PADDING = 0.1
LATENT = 128


def _leaky(x):
    return jnp.where(x >= 0, x, 0.2 * x)


def _bn2(x, g, b):
    m = jnp.mean(x, axis=(0, 2, 3), keepdims=True)
    v = jnp.var(x, axis=(0, 2, 3), keepdims=True)
    return (x - m) / jnp.sqrt(v + 1e-5) * g.reshape(1, -1, 1, 1) + b.reshape(1, -1, 1, 1)


def _bn1(x, g, b):
    m = jnp.mean(x, axis=(0, 2), keepdims=True)
    v = jnp.var(x, axis=(0, 2), keepdims=True)
    return (x - m) / jnp.sqrt(v + 1e-5) * g.reshape(1, -1, 1) + b.reshape(1, -1, 1)


def _conv2(x, W, g, b):
    return _leaky(_bn2(jnp.einsum('oc,bcnk->bonk', W, x), g, b))


def _conv1(x, W, g, b):
    return _leaky(_bn1(jnp.einsum('oc,bcn->bon', W, x), g, b))


def _knn_idx(x):
    inner = -2.0 * jnp.einsum('bcn,bcm->bnm', x, x)
    xx = jnp.sum(x ** 2, axis=1, keepdims=True)
    pd = -xx - inner - jnp.transpose(xx, (0, 2, 1))
    return jax.lax.top_k(pd, KNN)[1]


def _graph_feature(x):
    B, C, N = x.shape
    idx = _knn_idx(x) + (jnp.arange(B) * N).reshape(B, 1, 1)
    xt = jnp.transpose(x, (0, 2, 1)).reshape(B * N, C)
    feat = jnp.take(xt, idx.reshape(-1), axis=0).reshape(B, N, KNN, C)
    xr = xt.reshape(B, N, 1, C)
    out = jnp.concatenate([feat - xr, jnp.broadcast_to(xr, (B, N, KNN, C))], axis=3)
    return jnp.transpose(out, (0, 3, 1, 2))


def _div_kernel(s_ref, c_ref, o_ref):
    o_ref[...] = s_ref[...] / jnp.maximum(c_ref[...], 1.0)


def kernel(cloud, W0, g0, b0, W1, g1, b1, W2, g2, b2, W3, g3, b3, W4, g4, b4, W5, g5, b5, W6, g6, b6, W7, g7, b7):
    B, N, _ = cloud.shape
    x = jnp.transpose(cloud, (0, 2, 1))
    f = _conv2(_conv2(_graph_feature(x), W0, g0, b0), W1, g1, b1)
    f1 = jnp.max(f, axis=-1)
    f = _conv2(_conv2(_graph_feature(f1), W2, g2, b2), W3, g3, b3)
    f2 = jnp.max(f, axis=-1)
    f = _conv2(_graph_feature(f2), W4, g4, b4)
    f3 = jnp.max(f, axis=-1)
    cat = jnp.concatenate([f1, f2, f3], axis=1)
    emb = _conv1(cat, W5, g5, b5)
    gmax = jnp.max(emb, axis=-1, keepdims=True)
    gmax = jnp.broadcast_to(gmax, (B, gmax.shape[1], N))
    f = _conv1(jnp.concatenate([gmax, f1, f2, f3], axis=1), W6, g6, b6)
    f = _conv1(f, W7, g7, b7)
    feats = jnp.transpose(f, (0, 2, 1))
    p = jnp.clip(cloud / (1.0 + PADDING + 0.001) + 0.5, 0.0, 1.0 - 0.001)
    c = (p * GRID).astype(jnp.int32)
    idx = c[:, :, 0] + GRID * (c[:, :, 1] + GRID * c[:, :, 2])

    def scat(fb, ib):
        sums = jax.ops.segment_sum(fb, ib, num_segments=GRID ** 3)
        cnt = jax.ops.segment_sum(jnp.ones((fb.shape[0],), fb.dtype), ib, num_segments=GRID ** 3)
        return sums, cnt

    sums, cnt = jax.vmap(scat)(feats, idx)
    grid = pl.pallas_call(
        _div_kernel,
        grid=(B, 8),
        in_specs=[
            pl.BlockSpec((1, GRID ** 3 // 8, LATENT), lambda b, s: (b, s, 0)),
            pl.BlockSpec((1, GRID ** 3 // 8, LATENT), lambda b, s: (b, s, 0)),
        ],
        out_specs=pl.BlockSpec((1, GRID ** 3 // 8, LATENT), lambda b, s: (b, s, 0)),
        out_shape=jax.ShapeDtypeStruct((B, GRID ** 3, LATENT), jnp.float32),
    )(sums, jnp.broadcast_to(cnt[:, :, None], sums.shape))
    return jnp.transpose(grid, (0, 2, 1)).reshape(B, LATENT, GRID, GRID, GRID)


# speedup vs baseline: 3.6300x; 3.6300x over previous
import jax
import jax.numpy as jnp
from jax.experimental import pallas as pl

KNN = 20
GRID = 32
PADDING = 0.1
LATENT = 128


def _leaky(x):
    return jnp.where(x >= 0, x, 0.2 * x)


def _bn2(x, g, b):
    m = jnp.mean(x, axis=(0, 2, 3), keepdims=True)
    v = jnp.var(x, axis=(0, 2, 3), keepdims=True)
    return (x - m) / jnp.sqrt(v + 1e-5) * g.reshape(1, -1, 1, 1) + b.reshape(1, -1, 1, 1)


def _bn1(x, g, b):
    m = jnp.mean(x, axis=(0, 2), keepdims=True)
    v = jnp.var(x, axis=(0, 2), keepdims=True)
    return (x - m) / jnp.sqrt(v + 1e-5) * g.reshape(1, -1, 1) + b.reshape(1, -1, 1)


def _conv2(x, W, g, b):
    return _leaky(_bn2(jnp.einsum('oc,bcnk->bonk', W, x), g, b))


def _conv1(x, W, g, b):
    return _leaky(_bn1(jnp.einsum('oc,bcn->bon', W, x), g, b))


_KNN_ROWS = 512
_KNN_KPAD = 32


def _knn_kernel(xr_ref, xf_ref, o_ref):
    xr = xr_ref[0]
    xf = xf_ref[0]
    n = xf.shape[0]
    s = 2.0 * jax.lax.dot_general(
        xr, xf, (((1,), (1,)), ((), ())),
        preferred_element_type=jnp.float32)
    s = s - jnp.sum(xf * xf, axis=1)[None, :]
    col = jax.lax.broadcasted_iota(jnp.int32, s.shape, 1)
    neg = jnp.float32(-jnp.inf)

    def body(k, s):
        m = jnp.max(s, axis=1)
        hit = s == m[:, None]
        arg = jnp.min(jnp.where(hit, col, n), axis=1)
        o_ref[0, k, :] = arg
        return jnp.where(col == arg[:, None], neg, s)

    s = jax.lax.fori_loop(0, KNN, body, s)
    for k in range(KNN, _KNN_KPAD):
        o_ref[0, k, :] = jnp.zeros((xr.shape[0],), jnp.int32)


def _knn_idx(x):
    B, C, N = x.shape
    xt = jnp.transpose(x, (0, 2, 1))
    R = _KNN_ROWS
    out = pl.pallas_call(
        _knn_kernel,
        grid=(B, N // R),
        in_specs=[
            pl.BlockSpec((1, R, C), lambda b, r: (b, r, 0)),
            pl.BlockSpec((1, N, C), lambda b, r: (b, 0, 0)),
        ],
        out_specs=pl.BlockSpec((1, _KNN_KPAD, R), lambda b, r: (b, 0, r)),
        out_shape=jax.ShapeDtypeStruct((B, _KNN_KPAD, N), jnp.int32),
    )(xt, xt)
    return jnp.transpose(out[:, :KNN, :], (0, 2, 1))


def _graph_feature(x):
    B, C, N = x.shape
    idx = _knn_idx(x) + (jnp.arange(B) * N).reshape(B, 1, 1)
    xt = jnp.transpose(x, (0, 2, 1)).reshape(B * N, C)
    feat = jnp.take(xt, idx.reshape(-1), axis=0).reshape(B, N, KNN, C)
    xr = xt.reshape(B, N, 1, C)
    out = jnp.concatenate([feat - xr, jnp.broadcast_to(xr, (B, N, KNN, C))], axis=3)
    return jnp.transpose(out, (0, 3, 1, 2))


def _div_kernel(s_ref, c_ref, o_ref):
    o_ref[...] = s_ref[...] / jnp.maximum(c_ref[...], 1.0)


def kernel(cloud, W0, g0, b0, W1, g1, b1, W2, g2, b2, W3, g3, b3, W4, g4, b4, W5, g5, b5, W6, g6, b6, W7, g7, b7):
    B, N, _ = cloud.shape
    x = jnp.transpose(cloud, (0, 2, 1))
    f = _conv2(_conv2(_graph_feature(x), W0, g0, b0), W1, g1, b1)
    f1 = jnp.max(f, axis=-1)
    f = _conv2(_conv2(_graph_feature(f1), W2, g2, b2), W3, g3, b3)
    f2 = jnp.max(f, axis=-1)
    f = _conv2(_graph_feature(f2), W4, g4, b4)
    f3 = jnp.max(f, axis=-1)
    cat = jnp.concatenate([f1, f2, f3], axis=1)
    emb = _conv1(cat, W5, g5, b5)
    gmax = jnp.max(emb, axis=-1, keepdims=True)
    gmax = jnp.broadcast_to(gmax, (B, gmax.shape[1], N))
    f = _conv1(jnp.concatenate([gmax, f1, f2, f3], axis=1), W6, g6, b6)
    f = _conv1(f, W7, g7, b7)
    feats = jnp.transpose(f, (0, 2, 1))
    p = jnp.clip(cloud / (1.0 + PADDING + 0.001) + 0.5, 0.0, 1.0 - 0.001)
    c = (p * GRID).astype(jnp.int32)
    idx = c[:, :, 0] + GRID * (c[:, :, 1] + GRID * c[:, :, 2])

    def scat(fb, ib):
        sums = jax.ops.segment_sum(fb, ib, num_segments=GRID ** 3)
        cnt = jax.ops.segment_sum(jnp.ones((fb.shape[0],), fb.dtype), ib, num_segments=GRID ** 3)
        return sums, cnt

    sums, cnt = jax.vmap(scat)(feats, idx)
    grid = pl.pallas_call(
        _div_kernel,
        grid=(B, 8),
        in_specs=[
            pl.BlockSpec((1, GRID ** 3 // 8, LATENT), lambda b, s: (b, s, 0)),
            pl.BlockSpec((1, GRID ** 3 // 8, LATENT), lambda b, s: (b, s, 0)),
        ],
        out_specs=pl.BlockSpec((1, GRID ** 3 // 8, LATENT), lambda b, s: (b, s, 0)),
        out_shape=jax.ShapeDtypeStruct((B, GRID ** 3, LATENT), jnp.float32),
    )(sums, jnp.broadcast_to(cnt[:, :, None], sums.shape))
    return jnp.transpose(grid, (0, 2, 1)).reshape(B, LATENT, GRID, GRID, GRID)
